# baseline (speedup 1.0000x reference)
import functools

import jax
import jax.numpy as jnp
from jax import lax
from jax.experimental import pallas as pl
from jax.experimental.pallas import tpu as pltpu
from jax.experimental.pallas import tpu_sc as plsc

DIM = 64
NC = 2
NS = 16
NW = NC * NS
B_BLK = 128
NBUF = 4
SKEW_I = 3
SKEW_G = 2


@functools.lru_cache(maxsize=None)
def _gather_kernel(NROW, SEQ, V):
    n_blk = NROW // B_BLK
    n_tasks = SEQ * n_blk // NW
    assert n_tasks % NBUF == 0
    mesh = plsc.VectorSubcoreMesh(core_axis_name="c", subcore_axis_name="s")

    @functools.partial(
        pl.kernel,
        mesh=mesh,
        out_type=jax.ShapeDtypeStruct((SEQ, DIM // 8, n_blk, 8, B_BLK),
                                      jnp.float32),
        scratch_types=[
            pltpu.VMEM((NBUF, B_BLK), jnp.int32),
            pltpu.VMEM((NBUF, B_BLK, DIM), jnp.float32),
            pltpu.VMEM((NBUF, DIM // 8, 8, B_BLK), jnp.float32),
            pltpu.SemaphoreType.DMA((NBUF,)),
            pltpu.SemaphoreType.DMA((NBUF,)),
            pltpu.SemaphoreType.DMA((NBUF,)),
        ],
        compiler_params=pltpu.CompilerParams(use_tc_tiling_on_sc=False,
                                             needs_layout_passes=False),
    )
    def k(xt_hbm, table_hbm, out_hbm, idx_v, rows_v, trows_v,
          sem_i, sem_g, sem_o):
        wid = lax.axis_index("s") * NC + lax.axis_index("c")
        base = wid * n_tasks

        lane = lax.iota(jnp.int32, 16)
        row_idx = [lane + 16 * bb for bb in range(B_BLK // 16)]

        def coords(t):
            T = base + t
            return T // n_blk, (T % n_blk) * B_BLK

        def start_idx(t, slot):
            s, b0 = coords(t)
            pltpu.async_copy(xt_hbm.at[s, pl.ds(b0, B_BLK)], idx_v.at[slot],
                             sem_i.at[slot])

        def wait_idx(t, slot):
            s, b0 = coords(t)
            pltpu.make_async_copy(xt_hbm.at[s, pl.ds(b0, B_BLK)],
                                  idx_v.at[slot], sem_i.at[slot]).wait()

        def start_gather(slot):
            pltpu.async_copy(table_hbm.at[idx_v.at[slot]], rows_v.at[slot],
                             sem_g.at[slot])

        def wait_gather(slot):
            pltpu.make_async_copy(table_hbm.at[idx_v.at[slot]],
                                  rows_v.at[slot], sem_g.at[slot]).wait()

        def transpose(slot):
            @plsc.parallel_loop(0, DIM, step=8, unroll=2)
            def dbody(d0):
                dt = lax.shift_right_logical(d0, 3)
                for dw in range(8):
                    col = jnp.full((16,), d0 + dw, jnp.int32)
                    for bb in range(B_BLK // 16):
                        vec = plsc.load_gather(rows_v.at[slot],
                                               [row_idx[bb], col])
                        trows_v[slot, dt, dw, pl.ds(16 * bb, 16)] = vec

        def start_out(t, slot):
            s, b0 = coords(t)
            pltpu.async_copy(trows_v.at[slot],
                             out_hbm.at[s, :, b0 // B_BLK, :, :],
                             sem_o.at[slot])

        def wait_out(t, slot):
            s, b0 = coords(t)
            pltpu.make_async_copy(trows_v.at[slot],
                                  out_hbm.at[s, :, b0 // B_BLK, :, :],
                                  sem_o.at[slot]).wait()

        for t in range(SKEW_I):
            start_idx(t, t % NBUF)
        for t in range(SKEW_G):
            wait_idx(t, t % NBUF)
            start_gather(t % NBUF)

        def body(g, carry):
            for b in range(NBUF):
                t = g * NBUF + b
                s_i = (b + SKEW_I) % NBUF

                @pl.when(t + SKEW_I < n_tasks)
                def _():
                    start_idx(t + SKEW_I, s_i)

                s_g = (b + SKEW_G) % NBUF

                @pl.when(t + SKEW_G < n_tasks)
                def _():
                    wait_idx(t + SKEW_G, s_g)
                    start_gather(s_g)

                wait_gather(b)

                @pl.when(t >= NBUF)
                def _():
                    wait_out(t - NBUF, b)

                transpose(b)
                start_out(t, b)
            return carry

        lax.fori_loop(0, n_tasks // NBUF, body, 0)

        for b in range(NBUF):
            wait_out(n_tasks - NBUF + b, b)

    return k


@jax.jit
def kernel(x, table):
    xi = x.astype(jnp.int32)
    seq_pad = (x.shape[1] + 7) // 8 * 8
    xp = jnp.pad(xi, ((0, 0), (0, seq_pad - x.shape[1])))
    out5 = _gather_kernel(x.shape[0], x.shape[1], table.shape[0])(xp.T, table)
    return out5.transpose(2, 4, 0, 1, 3).reshape(x.shape[0], x.shape[1], DIM)


# speedup vs baseline: 1.8147x; 1.8147x over previous
import functools

import jax
import jax.numpy as jnp
from jax import lax
from jax.experimental import pallas as pl
from jax.experimental.pallas import tpu as pltpu
from jax.experimental.pallas import tpu_sc as plsc

DIM = 64
NC = 2
NS = 16
NW = NC * NS
B_BLK = 128
NBUF = 4
SKEW_I = 3
SKEW_G = 2


@functools.lru_cache(maxsize=None)
def _gather_kernel(NROW, SEQ, V):
    n_blk = NROW // B_BLK
    n_tasks = SEQ * n_blk // NW
    assert n_tasks % NBUF == 0
    mesh = plsc.VectorSubcoreMesh(core_axis_name="c", subcore_axis_name="s")

    @functools.partial(
        pl.kernel,
        mesh=mesh,
        out_type=jax.ShapeDtypeStruct((SEQ, DIM // 8, n_blk, 8, B_BLK),
                                      jnp.float32),
        scratch_types=[
            pltpu.VMEM((NBUF, B_BLK), jnp.int32),
            pltpu.VMEM((NBUF, B_BLK, DIM), jnp.float32),
            pltpu.VMEM((NBUF, DIM // 8, 8, B_BLK), jnp.float32),
            pltpu.SemaphoreType.DMA((NBUF,)),
            pltpu.SemaphoreType.DMA((NBUF,)),
            pltpu.SemaphoreType.DMA((NBUF,)),
        ],
        compiler_params=pltpu.CompilerParams(use_tc_tiling_on_sc=False,
                                             needs_layout_passes=False),
    )
    def k(xt_hbm, table_hbm, out_hbm, idx_v, rows_v, trows_v,
          sem_i, sem_g, sem_o):
        wid = lax.axis_index("s") * NC + lax.axis_index("c")
        base = wid * n_tasks

        lane = lax.iota(jnp.int32, 16)
        row_idx = [lane + 16 * bb for bb in range(B_BLK // 16)]

        def coords(t):
            T = base + t
            return T // n_blk, (T % n_blk) * B_BLK

        def start_idx(t, slot):
            s, b0 = coords(t)
            pltpu.async_copy(xt_hbm.at[s, pl.ds(b0, B_BLK)], idx_v.at[slot],
                             sem_i.at[slot])

        def wait_idx(t, slot):
            s, b0 = coords(t)
            pltpu.make_async_copy(xt_hbm.at[s, pl.ds(b0, B_BLK)],
                                  idx_v.at[slot], sem_i.at[slot]).wait()

        def start_gather(slot):
            pltpu.async_copy(table_hbm.at[idx_v.at[slot]], rows_v.at[slot],
                             sem_g.at[slot])

        def wait_gather(slot):
            pltpu.make_async_copy(table_hbm.at[idx_v.at[slot]],
                                  rows_v.at[slot], sem_g.at[slot]).wait()

        def transpose(slot):
            @plsc.parallel_loop(0, DIM, unroll=4)
            def dbody(d0):
                col = lax.bitwise_and(d0 + lane, DIM - 1)
                dt = lax.shift_right_logical(col, 3)
                dw = lax.bitwise_and(col, 7)
                for bb in range(B_BLK // 16):
                    vec = plsc.load_gather(rows_v.at[slot],
                                           [row_idx[bb], col])
                    plsc.store_scatter(trows_v.at[slot],
                                       [dt, dw, row_idx[bb]], vec)

        def start_out(t, slot):
            s, b0 = coords(t)
            pltpu.async_copy(trows_v.at[slot],
                             out_hbm.at[s, :, b0 // B_BLK, :, :],
                             sem_o.at[slot])

        def wait_out(t, slot):
            s, b0 = coords(t)
            pltpu.make_async_copy(trows_v.at[slot],
                                  out_hbm.at[s, :, b0 // B_BLK, :, :],
                                  sem_o.at[slot]).wait()

        for t in range(SKEW_I):
            start_idx(t, t % NBUF)
        for t in range(SKEW_G):
            wait_idx(t, t % NBUF)
            start_gather(t % NBUF)

        def body(g, carry):
            for b in range(NBUF):
                t = g * NBUF + b
                s_i = (b + SKEW_I) % NBUF

                @pl.when(t + SKEW_I < n_tasks)
                def _():
                    start_idx(t + SKEW_I, s_i)

                s_g = (b + SKEW_G) % NBUF

                @pl.when(t + SKEW_G < n_tasks)
                def _():
                    wait_idx(t + SKEW_G, s_g)
                    start_gather(s_g)

                wait_gather(b)

                @pl.when(t >= NBUF)
                def _():
                    wait_out(t - NBUF, b)

                transpose(b)
                start_out(t, b)
            return carry

        lax.fori_loop(0, n_tasks // NBUF, body, 0)

        for b in range(NBUF):
            wait_out(n_tasks - NBUF + b, b)

    return k


@jax.jit
def kernel(x, table):
    xi = x.astype(jnp.int32)
    seq_pad = (x.shape[1] + 7) // 8 * 8
    xp = jnp.pad(xi, ((0, 0), (0, seq_pad - x.shape[1])))
    out5 = _gather_kernel(x.shape[0], x.shape[1], table.shape[0])(xp.T, table)
    return out5.transpose(2, 4, 0, 1, 3).reshape(x.shape[0], x.shape[1], DIM)
